# baseline (speedup 1.0000x reference)
import functools

import jax
import jax.numpy as jnp
from jax import lax
from jax.experimental import pallas as pl
from jax.experimental.pallas import tpu as pltpu
from jax.experimental.pallas import tpu_sc as plsc

B = 16384
D = 32
DP = 128
NC = 2
NS = 16
NW = NC * NS
BPW = B // NW
CHUNK = 128
NCH = BPW // CHUNK
QSTRIDE = 521

_mesh = plsc.VectorSubcoreMesh(core_axis_name="c", subcore_axis_name="s")


@functools.partial(
    pl.kernel,
    mesh=_mesh,
    out_type=jax.ShapeDtypeStruct((B,), jnp.float32),
    scratch_types=[
        pltpu.VMEM((NCH, CHUNK), jnp.int32),
        pltpu.VMEM((NCH, CHUNK), jnp.int32),
        pltpu.VMEM((2, CHUNK, DP), jnp.float32),
        pltpu.VMEM((2, CHUNK, DP), jnp.float32),
        pltpu.VMEM((BPW,), jnp.float32),
        pltpu.VMEM((16 * QSTRIDE,), jnp.float32),
        pltpu.SemaphoreType.DMA,
    ],
    compiler_params=pltpu.CompilerParams(needs_layout_passes=False),
)
def _mf_kernel(uids_hbm, iids_hbm, umem_hbm, imem_hbm, out_hbm,
               uidx_v, iidx_v, urows_v, irows_v, out_v, qT_v, sem):
    wid = lax.axis_index("s") * NC + lax.axis_index("c")
    base = wid * BPW

    for j in range(NCH):
        pltpu.sync_copy(uids_hbm.at[pl.ds(base + j * CHUNK, CHUNK)],
                        uidx_v.at[j])
        pltpu.sync_copy(iids_hbm.at[pl.ds(base + j * CHUNK, CHUNK)],
                        iidx_v.at[j])

    lane = lax.iota(jnp.int32, 16)
    qidx0 = lane * QSTRIDE

    def fire(j):
        return [pltpu.async_copy(umem_hbm.at[uidx_v.at[j]],
                                 urows_v.at[j % 2], sem),
                pltpu.async_copy(imem_hbm.at[iidx_v.at[j]],
                                 irows_v.at[j % 2], sem)]

    inflight = fire(0)
    for j in range(NCH):
        nxt = fire(j + 1) if j + 1 < NCH else []
        for cp in inflight:
            cp.wait()
        inflight = nxt

        def row_body(r, _, j=j):
            u0 = urows_v[j % 2, r, pl.ds(0, 16)]
            u1 = urows_v[j % 2, r, pl.ds(16, 16)]
            i0 = irows_v[j % 2, r, pl.ds(0, 16)]
            i1 = irows_v[j % 2, r, pl.ds(16, 16)]
            v = u0 * i0 + u1 * i1
            plsc.store_scatter(qT_v, [qidx0 + (j * CHUNK + r)], v)
            return 0

        lax.fori_loop(0, CHUNK, row_body, 0, unroll=8)

    def group_body(g, _):
        acc = qT_v[pl.ds(g * 16, 16)]
        for c in range(1, 16):
            acc = acc + qT_v[pl.ds(c * QSTRIDE + g * 16, 16)]
        out_v[pl.ds(g * 16, 16)] = acc
        return 0

    lax.fori_loop(0, BPW // 16, group_body, 0, unroll=2)

    pltpu.sync_copy(out_v, out_hbm.at[pl.ds(base, BPW)])


def kernel(userids, itemids, user_memory, item_memory):
    um = jnp.pad(user_memory, ((0, 0), (0, DP - D)))
    im = jnp.pad(item_memory, ((0, 0), (0, DP - D)))
    return _mf_kernel(userids.astype(jnp.int32), itemids.astype(jnp.int32),
                      um, im)
